# Initial kernel scaffold; baseline (speedup 1.0000x reference)
import jax
import jax.numpy as jnp
from jax.experimental import pallas as pl


def kernel(out, left, pos_right, neg_right, single_gamma):
    raise NotImplementedError("write your pallas kernel here")


# speedup vs baseline: 6.1694x; 6.1694x over previous
import functools

import jax
import jax.numpy as jnp
from jax import lax
from jax.experimental import pallas as pl
from jax.experimental.pallas import tpu as pltpu
from jax.experimental.pallas import tpu_sc as plsc

N_NODES = 10000
D_FEAT = 128
N_PAIRS = 320000

NC = 2
NS = 16
NW = NC * NS
P_W = N_PAIRS // NW
CHUNK = 80
NCHUNK = P_W // CHUNK
BLKS = CHUNK // 16


def _make_sc_kernel():
    mesh = plsc.VectorSubcoreMesh(core_axis_name="c", subcore_axis_name="s")

    @functools.partial(
        pl.kernel,
        mesh=mesh,
        compiler_params=pltpu.CompilerParams(needs_layout_passes=False),
        out_type=jax.ShapeDtypeStruct((NW, 16), jnp.float32),
        scratch_types=[
            pltpu.VMEM((P_W,), jnp.int32),
            pltpu.VMEM((P_W,), jnp.int32),
            pltpu.VMEM((P_W,), jnp.int32),
            pltpu.VMEM((CHUNK, D_FEAT), jnp.float32),
            pltpu.VMEM((CHUNK, D_FEAT), jnp.float32),
            pltpu.VMEM((CHUNK, D_FEAT), jnp.float32),
            pltpu.VMEM((16,), jnp.float32),
            pltpu.VMEM((16,), jnp.float32),
            pltpu.SemaphoreType.DMA,
            pltpu.SemaphoreType.DMA,
            pltpu.SemaphoreType.DMA,
        ],
    )
    def sc_loss(tab_hbm, left_hbm, pos_hbm, neg_hbm, gam_hbm, out_hbm,
                lidx, pidx, nidx, lrow, prow, nrow, gv, resv,
                sem_l, sem_p, sem_n):
        cid = lax.axis_index("c")
        sid = lax.axis_index("s")
        wid = sid * NC + cid
        base = wid * P_W

        pltpu.sync_copy(left_hbm.at[pl.ds(base, P_W)], lidx)
        pltpu.sync_copy(pos_hbm.at[pl.ds(base, P_W)], pidx)
        pltpu.sync_copy(neg_hbm.at[pl.ds(base, P_W)], nidx)
        pltpu.sync_copy(gam_hbm, gv)
        g0 = gv[...][0]
        zero16 = jnp.zeros((16,), jnp.float32)

        def chunk_body(ci, loss):
            off = ci * CHUNK
            cl = pltpu.async_copy(
                tab_hbm.at[lidx.at[pl.ds(off, CHUNK)]], lrow, sem_l)
            cp = pltpu.async_copy(
                tab_hbm.at[pidx.at[pl.ds(off, CHUNK)]], prow, sem_p)
            cn = pltpu.async_copy(
                tab_hbm.at[nidx.at[pl.ds(off, CHUNK)]], nrow, sem_n)
            cl.wait()
            cp.wait()
            cn.wait()

            def pair_body(p, loss):
                acc = zero16
                for c in range(D_FEAT // 16):
                    sl = pl.ds(c * 16, 16)
                    lv = lrow[p, sl]
                    pv = prow[p, sl]
                    nv = nrow[p, sl]
                    acc = acc + lv * (nv - pv)
                m = jnp.sum(acc) + g0
                return loss + jnp.maximum(m, 0.0)

            return lax.fori_loop(0, CHUNK, pair_body, loss, unroll=2)

        loss = lax.fori_loop(0, NCHUNK, chunk_body, jnp.float32(0.0))
        resv[...] = jnp.full((16,), loss, jnp.float32)
        pltpu.sync_copy(resv, out_hbm.at[wid])

    return sc_loss


_sc_loss = _make_sc_kernel()


def kernel(out, left, pos_right, neg_right, single_gamma):
    left = left.astype(jnp.int32)
    pos_right = pos_right.astype(jnp.int32)
    neg_right = neg_right.astype(jnp.int32)
    gam = jnp.full((16,), single_gamma, jnp.float32)
    partials = _sc_loss(out, left, pos_right, neg_right, gam)
    return jnp.sum(partials[:, 0]) / left.shape[0]


# speedup vs baseline: 10.6324x; 1.7234x over previous
import functools

import jax
import jax.numpy as jnp
from jax import lax
from jax.experimental import pallas as pl
from jax.experimental.pallas import tpu as pltpu
from jax.experimental.pallas import tpu_sc as plsc

N_NODES = 10000
D_FEAT = 128
N_PAIRS = 320000

NC = 2
NS = 16
NW = NC * NS
P_W = N_PAIRS // NW
CHUNK = 80
NCHUNK = P_W // CHUNK
BLKS = CHUNK // 16


def _make_sc_kernel():
    mesh = plsc.VectorSubcoreMesh(core_axis_name="c", subcore_axis_name="s")

    @functools.partial(
        pl.kernel,
        mesh=mesh,
        compiler_params=pltpu.CompilerParams(needs_layout_passes=False),
        out_type=jax.ShapeDtypeStruct((NW, 16), jnp.float32),
        scratch_types=[
            pltpu.VMEM((P_W,), jnp.int32),
            pltpu.VMEM((P_W,), jnp.int32),
            pltpu.VMEM((P_W,), jnp.int32),
            pltpu.VMEM((CHUNK, D_FEAT), jnp.float32),
            pltpu.VMEM((CHUNK, D_FEAT), jnp.float32),
            pltpu.VMEM((CHUNK, D_FEAT), jnp.float32),
            pltpu.VMEM((CHUNK, D_FEAT), jnp.float32),
            pltpu.VMEM((CHUNK, D_FEAT), jnp.float32),
            pltpu.VMEM((CHUNK, D_FEAT), jnp.float32),
            pltpu.VMEM((16,), jnp.float32),
            pltpu.VMEM((16,), jnp.float32),
            pltpu.SemaphoreType.DMA,
            pltpu.SemaphoreType.DMA,
        ],
    )
    def sc_loss(tab_hbm, left_hbm, pos_hbm, neg_hbm, gam_hbm, out_hbm,
                lidx, pidx, nidx, lrow0, prow0, nrow0, lrow1, prow1, nrow1,
                gv, resv, sem0, sem1):
        cid = lax.axis_index("c")
        sid = lax.axis_index("s")
        wid = sid * NC + cid
        base = wid * P_W

        pltpu.sync_copy(left_hbm.at[pl.ds(base, P_W)], lidx)
        pltpu.sync_copy(pos_hbm.at[pl.ds(base, P_W)], pidx)
        pltpu.sync_copy(neg_hbm.at[pl.ds(base, P_W)], nidx)
        pltpu.sync_copy(gam_hbm, gv)
        g0 = gv[...][0]
        zero16 = jnp.zeros((16,), jnp.float32)

        bufs = ((lrow0, prow0, nrow0, sem0), (lrow1, prow1, nrow1, sem1))

        def start(b, ci):
            lr, pr, nr, sem = bufs[b]
            off = ci * CHUNK
            pltpu.async_copy(tab_hbm.at[lidx.at[pl.ds(off, CHUNK)]], lr, sem)
            pltpu.async_copy(tab_hbm.at[pidx.at[pl.ds(off, CHUNK)]], pr, sem)
            pltpu.async_copy(tab_hbm.at[nidx.at[pl.ds(off, CHUNK)]], nr, sem)

        def wait(b):
            lr, pr, nr, sem = bufs[b]
            for dst in (lr, pr, nr):
                pltpu.make_async_copy(tab_hbm.at[pl.ds(0, CHUNK)], dst,
                                      sem).wait()

        def compute(b, loss):
            lr, pr, nr, _ = bufs[b]

            def pair_body(p, loss):
                acc = zero16
                for c in range(D_FEAT // 16):
                    sl = pl.ds(c * 16, 16)
                    acc = acc + lr[p, sl] * (nr[p, sl] - pr[p, sl])
                m = jnp.sum(acc) + g0
                return loss + jnp.maximum(m, 0.0)

            return plsc.parallel_loop(0, CHUNK, carry=loss,
                                      unroll=2)(pair_body)

        start(0, 0)

        def body(i, loss):
            c0 = 2 * i
            start(1, c0 + 1)
            wait(0)
            loss = compute(0, loss)
            start(0, c0 + 2)
            wait(1)
            return compute(1, loss)

        loss = lax.fori_loop(0, (NCHUNK - 1) // 2, body, jnp.float32(0.0))
        wait(0)
        loss = compute(0, loss)
        resv[...] = jnp.full((16,), loss, jnp.float32)
        pltpu.sync_copy(resv, out_hbm.at[wid])

    return sc_loss


_sc_loss = _make_sc_kernel()


def kernel(out, left, pos_right, neg_right, single_gamma):
    left = left.astype(jnp.int32)
    pos_right = pos_right.astype(jnp.int32)
    neg_right = neg_right.astype(jnp.int32)
    gam = jnp.full((16,), single_gamma, jnp.float32)
    partials = _sc_loss(out, left, pos_right, neg_right, gam)
    return jnp.sum(partials[:, 0]) / left.shape[0]
